# baseline (speedup 1.0000x reference)
import jax
import jax.numpy as jnp
from jax import lax
from jax.experimental import pallas as pl
from jax.experimental.pallas import tpu as pltpu
from jax.experimental.pallas import tpu_sc as plsc

N_NODES = 10000
N_EDGES = 320000
HIDDEN = 128
EDGE_DIM = 16

NC = 2
NS = 16
NW = NC * NS
CHUNK = 80
E_PAD = 327680
PER_TILE = E_PAD // NW
NCHUNK = PER_TILE // CHUNK
AGG_ROWS = 10240
ROWS_PER_TILE = AGG_ROWS // NS


def _edge_mlp_body(a_ref, w_ref, b_ref, idx_ref, o_ref, src_ref, dst_ref):
    o_ref[...] = (
        jnp.dot(a_ref[...], w_ref[...], preferred_element_type=jnp.float32)
        + b_ref[...]
    )
    be = src_ref.shape[0]
    eid = pl.program_id(0) * be + lax.broadcasted_iota(jnp.int32, (be,), 0)
    valid = eid < N_EDGES
    src_ref[...] = jnp.where(valid, idx_ref[0, :], 0)
    dst_ref[...] = jnp.where(valid, idx_ref[1, :], N_NODES + (eid & 127))


def _edge_mlp(edge_attr, W_e, b_e, edge_index):
    be = 8192
    grid = E_PAD // be
    return pl.pallas_call(
        _edge_mlp_body,
        grid=(grid,),
        in_specs=[
            pl.BlockSpec((be, EDGE_DIM), lambda i: (i, 0)),
            pl.BlockSpec((EDGE_DIM, HIDDEN), lambda i: (0, 0)),
            pl.BlockSpec((1, HIDDEN), lambda i: (0, 0)),
            pl.BlockSpec((2, be), lambda i: (0, i)),
        ],
        out_specs=[
            pl.BlockSpec((be, HIDDEN), lambda i: (i, 0)),
            pl.BlockSpec((be,), lambda i: (i,)),
            pl.BlockSpec((be,), lambda i: (i,)),
        ],
        out_shape=[
            jax.ShapeDtypeStruct((E_PAD, HIDDEN), jnp.float32),
            jax.ShapeDtypeStruct((E_PAD,), jnp.int32),
            jax.ShapeDtypeStruct((E_PAD,), jnp.int32),
        ],
    )(edge_attr, W_e, b_e.reshape(1, HIDDEN), edge_index)


def _sc_body(x_hbm, src_hbm, dst_hbm, e_hbm, out_hbm,
             srcb0, srcb1, dstb0, dstb1, ebuf0, ebuf1, xbuf0, xbuf1,
             agg, esem0, esem1, isem0, isem1):
    cid = lax.axis_index("c")
    sid = lax.axis_index("s")
    wid = cid * NS + sid
    base = wid * PER_TILE

    zero16 = jnp.zeros((16,), jnp.float32)

    def zfill(i, _):
        for k in range(HIDDEN // 16):
            ebuf0[i, pl.ds(k * 16, 16)] = zero16
        return 0

    lax.fori_loop(0, CHUNK, zfill, 0)
    for t in range(ROWS_PER_TILE // CHUNK):
        pltpu.sync_copy(
            ebuf0, agg.at[pl.ds(sid * ROWS_PER_TILE + t * CHUNK, CHUNK)]
        )
    plsc.subcore_barrier()

    bufs = (
        (ebuf0, xbuf0, srcb0, dstb0, esem0, isem0),
        (ebuf1, xbuf1, srcb1, dstb1, esem1, isem1),
    )

    def issue_idx(c, b):
        _, _, sb, db, _, isem = bufs[b]
        off = base + c * CHUNK
        pltpu.async_copy(src_hbm.at[pl.ds(off, CHUNK)], sb, isem)
        pltpu.async_copy(dst_hbm.at[pl.ds(off, CHUNK)], db, isem)

    def wait_idx(b):
        _, _, sb, db, _, isem = bufs[b]
        pltpu.make_async_copy(src_hbm.at[pl.ds(0, CHUNK)], sb, isem).wait()
        pltpu.make_async_copy(src_hbm.at[pl.ds(0, CHUNK)], db, isem).wait()

    def issue_data(c, b):
        eb, xb, sb, _, esem, _ = bufs[b]
        pltpu.async_copy(e_hbm.at[pl.ds(base + c * CHUNK, CHUNK)], eb, esem)
        pltpu.async_copy(x_hbm.at[sb], xb, esem)

    def process(c, b):
        eb, xb, _, db, esem, _ = bufs[b]
        pltpu.make_async_copy(e_hbm.at[pl.ds(0, CHUNK)], eb, esem).wait()
        pltpu.make_async_copy(e_hbm.at[pl.ds(0, CHUNK)], xb, esem).wait()

        def erow(i, _):
            for k in range(HIDDEN // 16):
                sl = pl.ds(k * 16, 16)
                eb[i, sl] = jnp.maximum(eb[i, sl] + xb[i, sl], 0.0)
            return 0

        lax.fori_loop(0, CHUNK, erow, 0)
        pltpu.sync_copy(eb, agg.at[db], add=True)

    issue_idx(0, 0)
    issue_idx(1, 1)
    wait_idx(0)
    issue_data(0, 0)

    def half(c, b):
        @pl.when(c + 1 < NCHUNK)
        def _():
            wait_idx(1 - b)
            issue_data(c + 1, 1 - b)

        process(c, b)

        @pl.when(c + 2 < NCHUNK)
        def _():
            issue_idx(c + 2, b)

    def pair_body(i, _):
        half(i * 2, 0)
        half(i * 2 + 1, 1)
        return 0

    lax.fori_loop(0, NCHUNK // 2, pair_body, 0)
    plsc.subcore_barrier()

    rb = pl.ds(sid * ROWS_PER_TILE, ROWS_PER_TILE)
    pltpu.sync_copy(agg.at[rb], out_hbm.at[cid].at[rb])


def _sc_aggregate(x, src3, dst3, e):
    mesh = plsc.VectorSubcoreMesh(core_axis_name="c", subcore_axis_name="s")
    k = pl.kernel(
        _sc_body,
        out_type=jax.ShapeDtypeStruct((NC, AGG_ROWS, HIDDEN), jnp.float32),
        mesh=mesh,
        scratch_types=[
            pltpu.VMEM((CHUNK,), jnp.int32),
            pltpu.VMEM((CHUNK,), jnp.int32),
            pltpu.VMEM((CHUNK,), jnp.int32),
            pltpu.VMEM((CHUNK,), jnp.int32),
            pltpu.VMEM((CHUNK, HIDDEN), jnp.float32),
            pltpu.VMEM((CHUNK, HIDDEN), jnp.float32),
            pltpu.VMEM((CHUNK, HIDDEN), jnp.float32),
            pltpu.VMEM((CHUNK, HIDDEN), jnp.float32),
            pltpu.VMEM_SHARED((AGG_ROWS, HIDDEN), jnp.float32),
            pltpu.SemaphoreType.DMA,
            pltpu.SemaphoreType.DMA,
            pltpu.SemaphoreType.DMA,
            pltpu.SemaphoreType.DMA,
        ],
    )
    return k(x, src3, dst3, e)


def _node_mlp_body(x_ref, a_ref, w1_ref, b1_ref, w2_ref, b2_ref,
                   g_ref, bt_ref, o_ref):
    x = x_ref[...]
    h = x + a_ref[0] + a_ref[1]
    h = jnp.maximum(
        jnp.dot(h, w1_ref[...], preferred_element_type=jnp.float32)
        + b1_ref[...], 0.0)
    h = (jnp.dot(h, w2_ref[...], preferred_element_type=jnp.float32)
         + b2_ref[...])
    mean = jnp.mean(h, axis=0, keepdims=True)
    var = jnp.mean((h - mean) ** 2, axis=0, keepdims=True)
    h = (h - mean) * lax.rsqrt(var + 1e-5) * g_ref[...] + bt_ref[...]
    o_ref[...] = jnp.maximum(h, 0.0) + x


def _node_mlp(x, aggs, W1, b1, W2, b2, gamma, beta):
    return pl.pallas_call(
        _node_mlp_body,
        grid=(1,),
        in_specs=[
            pl.BlockSpec((N_NODES, HIDDEN), lambda i: (0, 0)),
            pl.BlockSpec((NC, N_NODES, HIDDEN), lambda i: (0, 0, 0)),
            pl.BlockSpec((HIDDEN, HIDDEN), lambda i: (0, 0)),
            pl.BlockSpec((1, HIDDEN), lambda i: (0, 0)),
            pl.BlockSpec((HIDDEN, HIDDEN), lambda i: (0, 0)),
            pl.BlockSpec((1, HIDDEN), lambda i: (0, 0)),
            pl.BlockSpec((1, HIDDEN), lambda i: (0, 0)),
            pl.BlockSpec((1, HIDDEN), lambda i: (0, 0)),
        ],
        out_specs=pl.BlockSpec((N_NODES, HIDDEN), lambda i: (0, 0)),
        out_shape=jax.ShapeDtypeStruct((N_NODES, HIDDEN), jnp.float32),
    )(x, aggs, W1, b1.reshape(1, HIDDEN), W2, b2.reshape(1, HIDDEN),
      gamma.reshape(1, HIDDEN), beta.reshape(1, HIDDEN))


def kernel(x, edge_index, edge_attr, W_e, b_e, W1, b1, W2, b2, gamma, beta):
    e, src, dst = _edge_mlp(edge_attr, W_e, b_e,
                            edge_index.astype(jnp.int32))
    aggs = _sc_aggregate(x, src, dst, e)
    return _node_mlp(x, aggs, W1, b1, W2, b2, gamma, beta)


# speedup vs baseline: 1.6885x; 1.6885x over previous
import jax
import jax.numpy as jnp
from jax import lax
from jax.experimental import pallas as pl
from jax.experimental.pallas import tpu as pltpu
from jax.experimental.pallas import tpu_sc as plsc

N_NODES = 10000
N_EDGES = 320000
HIDDEN = 128
EDGE_DIM = 16

NC = 2
NS = 16
NW = NC * NS
CHUNK = 80
E_PAD = 327680
PER_TILE = E_PAD // NW
NCHUNK = PER_TILE // CHUNK
AGG_ROWS = 10240
ROWS_PER_TILE = AGG_ROWS // NS


def _edge_mlp_body(a_ref, w_ref, b_ref, idx_ref, o_ref, src_ref, dst_ref):
    o_ref[...] = (
        jnp.dot(a_ref[...], w_ref[...], preferred_element_type=jnp.float32)
        + b_ref[...]
    )
    be = src_ref.shape[0]
    eid = pl.program_id(0) * be + lax.broadcasted_iota(jnp.int32, (be,), 0)
    valid = eid < N_EDGES
    src_ref[...] = jnp.where(valid, idx_ref[0, :], eid & 8191)
    dst_ref[...] = jnp.where(valid, idx_ref[1, :], N_NODES + (eid & 127))


def _edge_mlp(edge_attr, W_e, b_e, edge_index):
    be = 8192
    grid = E_PAD // be
    return pl.pallas_call(
        _edge_mlp_body,
        grid=(grid,),
        in_specs=[
            pl.BlockSpec((be, EDGE_DIM), lambda i: (i, 0)),
            pl.BlockSpec((EDGE_DIM, HIDDEN), lambda i: (0, 0)),
            pl.BlockSpec((1, HIDDEN), lambda i: (0, 0)),
            pl.BlockSpec((2, be), lambda i: (0, i)),
        ],
        out_specs=[
            pl.BlockSpec((be, HIDDEN), lambda i: (i, 0)),
            pl.BlockSpec((be,), lambda i: (i,)),
            pl.BlockSpec((be,), lambda i: (i,)),
        ],
        out_shape=[
            jax.ShapeDtypeStruct((E_PAD, HIDDEN), jnp.float32),
            jax.ShapeDtypeStruct((E_PAD,), jnp.int32),
            jax.ShapeDtypeStruct((E_PAD,), jnp.int32),
        ],
    )(edge_attr, W_e, b_e.reshape(1, HIDDEN), edge_index)


def _linearize_body(x_ref, o_ref):
    o_ref[...] = x_ref[...].reshape(o_ref.shape)


def _linearize(x):
    bn = 1000
    grid = N_NODES // bn
    return pl.pallas_call(
        _linearize_body,
        grid=(grid,),
        in_specs=[pl.BlockSpec((bn, HIDDEN), lambda i: (i, 0))],
        out_specs=pl.BlockSpec((bn * HIDDEN,), lambda i: (i,)),
        out_shape=jax.ShapeDtypeStruct((N_NODES * HIDDEN,), jnp.float32),
    )(x)


def _sc_body(x_hbm, src_hbm, dst_hbm, e_hbm, out_hbm,
             srcb0, srcb1, dstb0, dstb1, ebuf0, ebuf1, xbuf0, xbuf1,
             agg, esem0, esem1, isem0, isem1):
    cid = lax.axis_index("c")
    sid = lax.axis_index("s")
    wid = cid * NS + sid
    base = wid * PER_TILE

    zero16 = jnp.zeros((16,), jnp.float32)

    def zfill(i, _):
        for k in range(HIDDEN // 16):
            ebuf0[i, pl.ds(k * 16, 16)] = zero16
        return 0

    lax.fori_loop(0, CHUNK, zfill, 0)
    for t in range(ROWS_PER_TILE // CHUNK):
        pltpu.sync_copy(
            ebuf0, agg.at[pl.ds(sid * ROWS_PER_TILE + t * CHUNK, CHUNK)]
        )
    plsc.subcore_barrier()

    bufs = (
        (ebuf0, xbuf0, srcb0, dstb0, esem0, isem0),
        (ebuf1, xbuf1, srcb1, dstb1, esem1, isem1),
    )

    def issue_idx(c, b):
        _, _, sb, db, _, isem = bufs[b]
        off = base + c * CHUNK
        pltpu.async_copy(src_hbm.at[pl.ds(off, CHUNK)], sb, isem)
        pltpu.async_copy(dst_hbm.at[pl.ds(off, CHUNK)], db, isem)

    def wait_idx(b):
        _, _, sb, db, _, isem = bufs[b]
        pltpu.make_async_copy(src_hbm.at[pl.ds(0, CHUNK)], sb, isem).wait()
        pltpu.make_async_copy(src_hbm.at[pl.ds(0, CHUNK)], db, isem).wait()

    def issue_data(c, b):
        eb, xb, sb, _, esem, _ = bufs[b]
        pltpu.async_copy(e_hbm.at[pl.ds(base + c * CHUNK, CHUNK)], eb, esem)
        pltpu.async_copy(x_hbm.at[sb], xb, esem)

    def process(c, b):
        eb, xb, _, db, esem, _ = bufs[b]
        pltpu.make_async_copy(e_hbm.at[pl.ds(0, CHUNK)], eb, esem).wait()
        pltpu.make_async_copy(e_hbm.at[pl.ds(0, CHUNK)], xb, esem).wait()

        def erow(i, _):
            for k in range(HIDDEN // 16):
                sl = pl.ds(k * 16, 16)
                eb[i, sl] = jnp.maximum(eb[i, sl] + xb[i, sl], 0.0)
            return 0

        lax.fori_loop(0, CHUNK, erow, 0)
        pltpu.sync_copy(eb, agg.at[db], add=True)

    issue_idx(0, 0)
    issue_idx(1, 1)
    wait_idx(0)
    issue_data(0, 0)

    def half(c, b):
        @pl.when(c + 1 < NCHUNK)
        def _():
            wait_idx(1 - b)
            issue_data(c + 1, 1 - b)

        process(c, b)

        @pl.when(c + 2 < NCHUNK)
        def _():
            issue_idx(c + 2, b)

    def pair_body(i, _):
        half(i * 2, 0)
        half(i * 2 + 1, 1)
        return 0

    lax.fori_loop(0, NCHUNK // 2, pair_body, 0)
    plsc.subcore_barrier()

    rb = pl.ds(sid * ROWS_PER_TILE, ROWS_PER_TILE)
    pltpu.sync_copy(agg.at[rb], out_hbm.at[cid].at[rb])


def _sc_aggregate(x, src3, dst3, e):
    mesh = plsc.VectorSubcoreMesh(core_axis_name="c", subcore_axis_name="s")
    k = pl.kernel(
        _sc_body,
        out_type=jax.ShapeDtypeStruct((NC, AGG_ROWS, HIDDEN), jnp.float32),
        mesh=mesh,
        scratch_types=[
            pltpu.VMEM((CHUNK,), jnp.int32),
            pltpu.VMEM((CHUNK,), jnp.int32),
            pltpu.VMEM((CHUNK,), jnp.int32),
            pltpu.VMEM((CHUNK,), jnp.int32),
            pltpu.VMEM((CHUNK, HIDDEN), jnp.float32),
            pltpu.VMEM((CHUNK, HIDDEN), jnp.float32),
            pltpu.VMEM((CHUNK, HIDDEN), jnp.float32),
            pltpu.VMEM((CHUNK, HIDDEN), jnp.float32),
            pltpu.VMEM_SHARED((AGG_ROWS, HIDDEN), jnp.float32),
            pltpu.SemaphoreType.DMA,
            pltpu.SemaphoreType.DMA,
            pltpu.SemaphoreType.DMA,
            pltpu.SemaphoreType.DMA,
        ],
    )
    return k(x, src3, dst3, e)


def _node_mlp_body(x_ref, a_ref, w1_ref, b1_ref, w2_ref, b2_ref,
                   g_ref, bt_ref, o_ref):
    x = x_ref[...]
    h = x + a_ref[0] + a_ref[1]
    h = jnp.maximum(
        jnp.dot(h, w1_ref[...], preferred_element_type=jnp.float32)
        + b1_ref[...], 0.0)
    h = (jnp.dot(h, w2_ref[...], preferred_element_type=jnp.float32)
         + b2_ref[...])
    mean = jnp.mean(h, axis=0, keepdims=True)
    var = jnp.mean((h - mean) ** 2, axis=0, keepdims=True)
    h = (h - mean) * lax.rsqrt(var + 1e-5) * g_ref[...] + bt_ref[...]
    o_ref[...] = jnp.maximum(h, 0.0) + x


def _node_mlp(x, aggs, W1, b1, W2, b2, gamma, beta):
    return pl.pallas_call(
        _node_mlp_body,
        grid=(1,),
        in_specs=[
            pl.BlockSpec((N_NODES, HIDDEN), lambda i: (0, 0)),
            pl.BlockSpec((NC, N_NODES, HIDDEN), lambda i: (0, 0, 0)),
            pl.BlockSpec((HIDDEN, HIDDEN), lambda i: (0, 0)),
            pl.BlockSpec((1, HIDDEN), lambda i: (0, 0)),
            pl.BlockSpec((HIDDEN, HIDDEN), lambda i: (0, 0)),
            pl.BlockSpec((1, HIDDEN), lambda i: (0, 0)),
            pl.BlockSpec((1, HIDDEN), lambda i: (0, 0)),
            pl.BlockSpec((1, HIDDEN), lambda i: (0, 0)),
        ],
        out_specs=pl.BlockSpec((N_NODES, HIDDEN), lambda i: (0, 0)),
        out_shape=jax.ShapeDtypeStruct((N_NODES, HIDDEN), jnp.float32),
    )(x, aggs, W1, b1.reshape(1, HIDDEN), W2, b2.reshape(1, HIDDEN),
      gamma.reshape(1, HIDDEN), beta.reshape(1, HIDDEN))


def kernel(x, edge_index, edge_attr, W_e, b_e, W1, b1, W2, b2, gamma, beta):
    e, src, dst = _edge_mlp(edge_attr, W_e, b_e,
                            edge_index.astype(jnp.int32))
    x_lin = _linearize(x).reshape(N_NODES, HIDDEN)
    aggs = _sc_aggregate(x_lin, src, dst, e)
    return _node_mlp(x, aggs, W1, b1, W2, b2, gamma, beta)


# speedup vs baseline: 2.2563x; 1.3363x over previous
import jax
import jax.numpy as jnp
from jax import lax
from jax.experimental import pallas as pl
from jax.experimental.pallas import tpu as pltpu
from jax.experimental.pallas import tpu_sc as plsc

N_NODES = 10000
N_EDGES = 320000
HIDDEN = 128
EDGE_DIM = 16

NC = 2
NS = 16
NW = NC * NS
CHUNK = 80
E_PAD = 327680
PER_TILE = E_PAD // NW
NCHUNK = PER_TILE // CHUNK
AGG_ROWS = 10240
ROWS_PER_TILE = AGG_ROWS // NS


def _edge_mlp_body(at_ref, w_ref, b_ref, idx_ref, o_ref, src_ref, dst_ref):
    o_ref[...] = lax.dot_general(
        at_ref[...], w_ref[...], (((0,), (0,)), ((), ())),
        preferred_element_type=jnp.float32,
    ) + b_ref[...]
    be = src_ref.shape[0]
    eid = pl.program_id(0) * be + lax.broadcasted_iota(jnp.int32, (be,), 0)
    valid = eid < N_EDGES
    src_ref[...] = jnp.where(valid, idx_ref[0, :], eid & 8191)
    dst_ref[...] = jnp.where(valid, idx_ref[1, :], N_NODES + (eid & 127))


def _edge_mlp(edge_attr, W_e, b_e, edge_index):
    be = 8192
    grid = E_PAD // be
    return pl.pallas_call(
        _edge_mlp_body,
        grid=(grid,),
        in_specs=[
            pl.BlockSpec((EDGE_DIM, be), lambda i: (0, i)),
            pl.BlockSpec((EDGE_DIM, HIDDEN), lambda i: (0, 0)),
            pl.BlockSpec((1, HIDDEN), lambda i: (0, 0)),
            pl.BlockSpec((2, be), lambda i: (0, i)),
        ],
        out_specs=[
            pl.BlockSpec((be, HIDDEN), lambda i: (i, 0)),
            pl.BlockSpec((be,), lambda i: (i,)),
            pl.BlockSpec((be,), lambda i: (i,)),
        ],
        out_shape=[
            jax.ShapeDtypeStruct((E_PAD, HIDDEN), jnp.float32),
            jax.ShapeDtypeStruct((E_PAD,), jnp.int32),
            jax.ShapeDtypeStruct((E_PAD,), jnp.int32),
        ],
    )(edge_attr.T, W_e, b_e.reshape(1, HIDDEN), edge_index)


def _linearize_body(x_ref, o_ref):
    o_ref[...] = x_ref[...].reshape(o_ref.shape)


def _linearize(x):
    bn = 1000
    grid = N_NODES // bn
    return pl.pallas_call(
        _linearize_body,
        grid=(grid,),
        in_specs=[pl.BlockSpec((bn, HIDDEN), lambda i: (i, 0))],
        out_specs=pl.BlockSpec((bn * HIDDEN,), lambda i: (i,)),
        out_shape=jax.ShapeDtypeStruct((N_NODES * HIDDEN,), jnp.float32),
    )(x)


def _sc_body(x_hbm, src_hbm, dst_hbm, e_hbm, out_hbm,
             srcb0, srcb1, dstb0, dstb1, ebuf0, ebuf1, xbuf0, xbuf1,
             agg, esem0, esem1, isem0, isem1):
    cid = lax.axis_index("c")
    sid = lax.axis_index("s")
    wid = cid * NS + sid
    base = wid * PER_TILE

    zero16 = jnp.zeros((16,), jnp.float32)

    def zfill(i, _):
        for k in range(HIDDEN // 16):
            ebuf0[i, pl.ds(k * 16, 16)] = zero16
        return 0

    lax.fori_loop(0, CHUNK, zfill, 0)
    for t in range(ROWS_PER_TILE // CHUNK):
        pltpu.sync_copy(
            ebuf0, agg.at[pl.ds(sid * ROWS_PER_TILE + t * CHUNK, CHUNK)]
        )
    plsc.subcore_barrier()

    bufs = (
        (ebuf0, xbuf0, srcb0, dstb0, esem0, isem0),
        (ebuf1, xbuf1, srcb1, dstb1, esem1, isem1),
    )

    def issue_idx(c, b):
        _, _, sb, db, _, isem = bufs[b]
        off = base + c * CHUNK
        pltpu.async_copy(src_hbm.at[pl.ds(off, CHUNK)], sb, isem)
        pltpu.async_copy(dst_hbm.at[pl.ds(off, CHUNK)], db, isem)

    def wait_idx(b):
        _, _, sb, db, _, isem = bufs[b]
        pltpu.make_async_copy(src_hbm.at[pl.ds(0, CHUNK)], sb, isem).wait()
        pltpu.make_async_copy(src_hbm.at[pl.ds(0, CHUNK)], db, isem).wait()

    def issue_data(c, b):
        eb, xb, sb, _, esem, _ = bufs[b]
        pltpu.async_copy(e_hbm.at[pl.ds(base + c * CHUNK, CHUNK)], eb, esem)
        pltpu.async_copy(x_hbm.at[sb], xb, esem)

    def process(c, b):
        eb, xb, _, db, esem, _ = bufs[b]
        pltpu.make_async_copy(e_hbm.at[pl.ds(0, CHUNK)], eb, esem).wait()
        pltpu.make_async_copy(e_hbm.at[pl.ds(0, CHUNK)], xb, esem).wait()

        def erow(i, _):
            for k in range(HIDDEN // 16):
                sl = pl.ds(k * 16, 16)
                eb[i, sl] = jnp.maximum(eb[i, sl] + xb[i, sl], 0.0)
            return 0

        lax.fori_loop(0, CHUNK, erow, 0)
        pltpu.sync_copy(eb, agg.at[db], add=True)

    issue_idx(0, 0)
    issue_idx(1, 1)
    wait_idx(0)
    issue_data(0, 0)

    def half(c, b):
        @pl.when(c + 1 < NCHUNK)
        def _():
            wait_idx(1 - b)
            issue_data(c + 1, 1 - b)

        process(c, b)

        @pl.when(c + 2 < NCHUNK)
        def _():
            issue_idx(c + 2, b)

    def pair_body(i, _):
        half(i * 2, 0)
        half(i * 2 + 1, 1)
        return 0

    lax.fori_loop(0, NCHUNK // 2, pair_body, 0)
    plsc.subcore_barrier()

    rb = pl.ds(sid * ROWS_PER_TILE, ROWS_PER_TILE)
    pltpu.sync_copy(agg.at[rb], out_hbm.at[cid].at[rb])


def _sc_aggregate(x, src3, dst3, e):
    mesh = plsc.VectorSubcoreMesh(core_axis_name="c", subcore_axis_name="s")
    k = pl.kernel(
        _sc_body,
        out_type=jax.ShapeDtypeStruct((NC, AGG_ROWS, HIDDEN), jnp.float32),
        mesh=mesh,
        scratch_types=[
            pltpu.VMEM((CHUNK,), jnp.int32),
            pltpu.VMEM((CHUNK,), jnp.int32),
            pltpu.VMEM((CHUNK,), jnp.int32),
            pltpu.VMEM((CHUNK,), jnp.int32),
            pltpu.VMEM((CHUNK, HIDDEN), jnp.float32),
            pltpu.VMEM((CHUNK, HIDDEN), jnp.float32),
            pltpu.VMEM((CHUNK, HIDDEN), jnp.float32),
            pltpu.VMEM((CHUNK, HIDDEN), jnp.float32),
            pltpu.VMEM_SHARED((AGG_ROWS, HIDDEN), jnp.float32),
            pltpu.SemaphoreType.DMA,
            pltpu.SemaphoreType.DMA,
            pltpu.SemaphoreType.DMA,
            pltpu.SemaphoreType.DMA,
        ],
    )
    return k(x, src3, dst3, e)


def _node_mlp_body(x_ref, a_ref, w1_ref, b1_ref, w2_ref, b2_ref,
                   g_ref, bt_ref, o_ref):
    x = x_ref[...]
    h = x + a_ref[0] + a_ref[1]
    h = jnp.maximum(
        jnp.dot(h, w1_ref[...], preferred_element_type=jnp.float32)
        + b1_ref[...], 0.0)
    h = (jnp.dot(h, w2_ref[...], preferred_element_type=jnp.float32)
         + b2_ref[...])
    mean = jnp.mean(h, axis=0, keepdims=True)
    var = jnp.mean((h - mean) ** 2, axis=0, keepdims=True)
    h = (h - mean) * lax.rsqrt(var + 1e-5) * g_ref[...] + bt_ref[...]
    o_ref[...] = jnp.maximum(h, 0.0) + x


def _node_mlp(x, aggs, W1, b1, W2, b2, gamma, beta):
    return pl.pallas_call(
        _node_mlp_body,
        grid=(1,),
        in_specs=[
            pl.BlockSpec((N_NODES, HIDDEN), lambda i: (0, 0)),
            pl.BlockSpec((NC, N_NODES, HIDDEN), lambda i: (0, 0, 0)),
            pl.BlockSpec((HIDDEN, HIDDEN), lambda i: (0, 0)),
            pl.BlockSpec((1, HIDDEN), lambda i: (0, 0)),
            pl.BlockSpec((HIDDEN, HIDDEN), lambda i: (0, 0)),
            pl.BlockSpec((1, HIDDEN), lambda i: (0, 0)),
            pl.BlockSpec((1, HIDDEN), lambda i: (0, 0)),
            pl.BlockSpec((1, HIDDEN), lambda i: (0, 0)),
        ],
        out_specs=pl.BlockSpec((N_NODES, HIDDEN), lambda i: (0, 0)),
        out_shape=jax.ShapeDtypeStruct((N_NODES, HIDDEN), jnp.float32),
    )(x, aggs, W1, b1.reshape(1, HIDDEN), W2, b2.reshape(1, HIDDEN),
      gamma.reshape(1, HIDDEN), beta.reshape(1, HIDDEN))


def kernel(x, edge_index, edge_attr, W_e, b_e, W1, b1, W2, b2, gamma, beta):
    e, src, dst = _edge_mlp(edge_attr, W_e, b_e,
                            edge_index.astype(jnp.int32))
    x_lin = _linearize(x).reshape(N_NODES, HIDDEN)
    aggs = _sc_aggregate(x_lin, src, dst, e)
    return _node_mlp(x, aggs, W1, b1, W2, b2, gamma, beta)


# speedup vs baseline: 2.4726x; 1.0959x over previous
import jax
import jax.numpy as jnp
from jax import lax
from jax.experimental import pallas as pl
from jax.experimental.pallas import tpu as pltpu
from jax.experimental.pallas import tpu_sc as plsc

N_NODES = 10000
N_EDGES = 320000
HIDDEN = 128
EDGE_DIM = 16

NC = 2
NS = 16
NW = NC * NS
CHUNK = 80
E_PAD = 327680
PER_TILE = E_PAD // NW
NCHUNK = PER_TILE // CHUNK
AGG_ROWS = 10240
ROWS_PER_TILE = AGG_ROWS // NS


def _edge_mlp_body(at_ref, w_ref, b_ref, idx_ref, o_ref, src_ref, dst_ref):
    o_ref[...] = lax.dot_general(
        at_ref[...], w_ref[...], (((0,), (0,)), ((), ())),
        preferred_element_type=jnp.float32,
    ) + b_ref[...]
    be = src_ref.shape[0]
    eid = pl.program_id(0) * be + lax.broadcasted_iota(jnp.int32, (be,), 0)
    valid = eid < N_EDGES
    src_ref[...] = jnp.where(valid, idx_ref[0, :], eid & 8191)
    dst_ref[...] = jnp.where(valid, idx_ref[1, :], N_NODES + (eid & 127))


def _edge_mlp(edge_attr, W_e, b_e, edge_index):
    be = 8192
    grid = E_PAD // be
    return pl.pallas_call(
        _edge_mlp_body,
        grid=(grid,),
        in_specs=[
            pl.BlockSpec((EDGE_DIM, be), lambda i: (0, i)),
            pl.BlockSpec((EDGE_DIM, HIDDEN), lambda i: (0, 0)),
            pl.BlockSpec((1, HIDDEN), lambda i: (0, 0)),
            pl.BlockSpec((2, be), lambda i: (0, i)),
        ],
        out_specs=[
            pl.BlockSpec((be, HIDDEN), lambda i: (i, 0)),
            pl.BlockSpec((be,), lambda i: (i,)),
            pl.BlockSpec((be,), lambda i: (i,)),
        ],
        out_shape=[
            jax.ShapeDtypeStruct((E_PAD, HIDDEN), jnp.float32),
            jax.ShapeDtypeStruct((E_PAD,), jnp.int32),
            jax.ShapeDtypeStruct((E_PAD,), jnp.int32),
        ],
    )(edge_attr.T, W_e, b_e.reshape(1, HIDDEN), edge_index)


def _linearize_body(x_ref, o_ref):
    o_ref[...] = x_ref[...].reshape(o_ref.shape)


def _linearize(x):
    bn = 1000
    grid = N_NODES // bn
    return pl.pallas_call(
        _linearize_body,
        grid=(grid,),
        in_specs=[pl.BlockSpec((bn, HIDDEN), lambda i: (i, 0))],
        out_specs=pl.BlockSpec((bn * HIDDEN,), lambda i: (i,)),
        out_shape=jax.ShapeDtypeStruct((N_NODES * HIDDEN,), jnp.float32),
    )(x)


def _sc_body(x_hbm, src_hbm, dst_hbm, e_hbm, out_hbm,
             srcb0, srcb1, dstb0, dstb1, ebuf0, ebuf1, xbuf0, xbuf1,
             agg, esem0, esem1, isem0, isem1, ssem0, ssem1):
    cid = lax.axis_index("c")
    sid = lax.axis_index("s")
    wid = cid * NS + sid
    base = wid * PER_TILE

    zero16 = jnp.zeros((16,), jnp.float32)

    def zfill(i, _):
        for k in range(HIDDEN // 16):
            ebuf0[i, pl.ds(k * 16, 16)] = zero16
        return 0

    lax.fori_loop(0, CHUNK, zfill, 0)
    for t in range(ROWS_PER_TILE // CHUNK):
        pltpu.sync_copy(
            ebuf0, agg.at[pl.ds(sid * ROWS_PER_TILE + t * CHUNK, CHUNK)]
        )
    plsc.subcore_barrier()

    bufs = (
        (ebuf0, xbuf0, srcb0, dstb0, esem0, isem0, ssem0),
        (ebuf1, xbuf1, srcb1, dstb1, esem1, isem1, ssem1),
    )

    def issue_idx(c, b):
        sb, db, isem = bufs[b][2], bufs[b][3], bufs[b][5]
        off = base + c * CHUNK
        pltpu.async_copy(src_hbm.at[pl.ds(off, CHUNK)], sb, isem)
        pltpu.async_copy(dst_hbm.at[pl.ds(off, CHUNK)], db, isem)

    def wait_idx(b):
        sb, db, isem = bufs[b][2], bufs[b][3], bufs[b][5]
        pltpu.make_async_copy(src_hbm.at[pl.ds(0, CHUNK)], sb, isem).wait()
        pltpu.make_async_copy(src_hbm.at[pl.ds(0, CHUNK)], db, isem).wait()

    def issue_data(c, b):
        eb, xb, sb, esem = bufs[b][0], bufs[b][1], bufs[b][2], bufs[b][4]
        pltpu.async_copy(e_hbm.at[pl.ds(base + c * CHUNK, CHUNK)], eb, esem)
        pltpu.async_copy(x_hbm.at[sb], xb, esem)

    def wait_scatter(b):
        eb, db, ssem = bufs[b][0], bufs[b][3], bufs[b][6]
        pltpu.make_async_copy(eb, agg.at[db], ssem).wait()

    def process(c, b):
        eb, xb, db, esem, ssem = (bufs[b][0], bufs[b][1], bufs[b][3],
                                  bufs[b][4], bufs[b][6])
        pltpu.make_async_copy(e_hbm.at[pl.ds(0, CHUNK)], eb, esem).wait()
        pltpu.make_async_copy(e_hbm.at[pl.ds(0, CHUNK)], xb, esem).wait()

        def erow(i, _):
            for k in range(HIDDEN // 16):
                sl = pl.ds(k * 16, 16)
                eb[i, sl] = jnp.maximum(eb[i, sl] + xb[i, sl], 0.0)
            return 0

        lax.fori_loop(0, CHUNK, erow, 0)
        pltpu.async_copy(eb, agg.at[db], ssem, add=True)

    issue_idx(0, 0)
    issue_idx(1, 1)
    wait_idx(0)
    issue_data(0, 0)

    def half(c, b):
        @pl.when(jnp.logical_and(c + 1 < NCHUNK, c >= 1))
        def _():
            wait_scatter(1 - b)

        @pl.when(c + 1 < NCHUNK)
        def _():
            wait_idx(1 - b)
            issue_data(c + 1, 1 - b)

        process(c, b)

        @pl.when(c + 2 < NCHUNK)
        def _():
            issue_idx(c + 2, b)

    def pair_body(i, _):
        half(i * 2, 0)
        half(i * 2 + 1, 1)
        return 0

    lax.fori_loop(0, NCHUNK // 2, pair_body, 0)
    wait_scatter(0)
    wait_scatter(1)
    plsc.subcore_barrier()

    rb = pl.ds(sid * ROWS_PER_TILE, ROWS_PER_TILE)
    pltpu.sync_copy(agg.at[rb], out_hbm.at[cid].at[rb])


def _sc_aggregate(x, src3, dst3, e):
    mesh = plsc.VectorSubcoreMesh(core_axis_name="c", subcore_axis_name="s")
    k = pl.kernel(
        _sc_body,
        out_type=jax.ShapeDtypeStruct((NC, AGG_ROWS, HIDDEN), jnp.float32),
        mesh=mesh,
        scratch_types=[
            pltpu.VMEM((CHUNK,), jnp.int32),
            pltpu.VMEM((CHUNK,), jnp.int32),
            pltpu.VMEM((CHUNK,), jnp.int32),
            pltpu.VMEM((CHUNK,), jnp.int32),
            pltpu.VMEM((CHUNK, HIDDEN), jnp.float32),
            pltpu.VMEM((CHUNK, HIDDEN), jnp.float32),
            pltpu.VMEM((CHUNK, HIDDEN), jnp.float32),
            pltpu.VMEM((CHUNK, HIDDEN), jnp.float32),
            pltpu.VMEM_SHARED((AGG_ROWS, HIDDEN), jnp.float32),
            pltpu.SemaphoreType.DMA,
            pltpu.SemaphoreType.DMA,
            pltpu.SemaphoreType.DMA,
            pltpu.SemaphoreType.DMA,
            pltpu.SemaphoreType.DMA,
            pltpu.SemaphoreType.DMA,
        ],
    )
    return k(x, src3, dst3, e)


def _node_mlp_body(x_ref, a_ref, w1_ref, b1_ref, w2_ref, b2_ref,
                   g_ref, bt_ref, o_ref):
    x = x_ref[...]
    h = x + a_ref[0] + a_ref[1]
    h = jnp.maximum(
        jnp.dot(h, w1_ref[...], preferred_element_type=jnp.float32)
        + b1_ref[...], 0.0)
    h = (jnp.dot(h, w2_ref[...], preferred_element_type=jnp.float32)
         + b2_ref[...])
    mean = jnp.mean(h, axis=0, keepdims=True)
    var = jnp.mean((h - mean) ** 2, axis=0, keepdims=True)
    h = (h - mean) * lax.rsqrt(var + 1e-5) * g_ref[...] + bt_ref[...]
    o_ref[...] = jnp.maximum(h, 0.0) + x


def _node_mlp(x, aggs, W1, b1, W2, b2, gamma, beta):
    return pl.pallas_call(
        _node_mlp_body,
        grid=(1,),
        in_specs=[
            pl.BlockSpec((N_NODES, HIDDEN), lambda i: (0, 0)),
            pl.BlockSpec((NC, N_NODES, HIDDEN), lambda i: (0, 0, 0)),
            pl.BlockSpec((HIDDEN, HIDDEN), lambda i: (0, 0)),
            pl.BlockSpec((1, HIDDEN), lambda i: (0, 0)),
            pl.BlockSpec((HIDDEN, HIDDEN), lambda i: (0, 0)),
            pl.BlockSpec((1, HIDDEN), lambda i: (0, 0)),
            pl.BlockSpec((1, HIDDEN), lambda i: (0, 0)),
            pl.BlockSpec((1, HIDDEN), lambda i: (0, 0)),
        ],
        out_specs=pl.BlockSpec((N_NODES, HIDDEN), lambda i: (0, 0)),
        out_shape=jax.ShapeDtypeStruct((N_NODES, HIDDEN), jnp.float32),
    )(x, aggs, W1, b1.reshape(1, HIDDEN), W2, b2.reshape(1, HIDDEN),
      gamma.reshape(1, HIDDEN), beta.reshape(1, HIDDEN))


def kernel(x, edge_index, edge_attr, W_e, b_e, W1, b1, W2, b2, gamma, beta):
    e, src, dst = _edge_mlp(edge_attr, W_e, b_e,
                            edge_index.astype(jnp.int32))
    x_lin = _linearize(x).reshape(N_NODES, HIDDEN)
    aggs = _sc_aggregate(x_lin, src, dst, e)
    return _node_mlp(x, aggs, W1, b1, W2, b2, gamma, beta)


# speedup vs baseline: 2.6531x; 1.0730x over previous
import jax
import jax.numpy as jnp
from jax import lax
from jax.experimental import pallas as pl
from jax.experimental.pallas import tpu as pltpu
from jax.experimental.pallas import tpu_sc as plsc

N_NODES = 10000
N_EDGES = 320000
HIDDEN = 128
EDGE_DIM = 16

NC = 2
NS = 16
NW = NC * NS
CHUNK = 80
E_PAD = 327680
PER_TILE = E_PAD // NW
NCHUNK = PER_TILE // CHUNK
HALF = PER_TILE // 2
NSUPER = NCHUNK // 2
SWORDS = CHUNK * HIDDEN
AGG_ROWS = 10240
ROWS_PER_TILE = AGG_ROWS // NS


def _edge_mlp_body(at_ref, w_ref, b_ref, idx_ref, o_ref, src_ref, dst_ref):
    e = lax.dot_general(
        at_ref[...], w_ref[...], (((0,), (0,)), ((), ())),
        preferred_element_type=jnp.float32,
    ) + b_ref[...]
    ra = e[:HALF].astype(jnp.bfloat16).astype(jnp.float32)
    rb = e[HALF:].astype(jnp.bfloat16).astype(jnp.float32)
    ia = lax.shift_right_logical(lax.bitcast_convert_type(ra, jnp.int32), 16)
    ib = lax.bitcast_convert_type(rb, jnp.int32) & jnp.int32(-65536)
    o_ref[...] = (ia | ib).reshape(o_ref.shape)
    be = src_ref.shape[0]
    eid = pl.program_id(0) * be + lax.broadcasted_iota(jnp.int32, (be,), 0)
    valid = eid < N_EDGES
    src_ref[...] = jnp.where(valid, idx_ref[0, :], eid & 8191)
    dst_ref[...] = jnp.where(valid, idx_ref[1, :], N_NODES + (eid & 127))


def _edge_mlp(edge_attr, W_e, b_e, edge_index):
    be = PER_TILE
    grid = E_PAD // be
    return pl.pallas_call(
        _edge_mlp_body,
        grid=(grid,),
        in_specs=[
            pl.BlockSpec((EDGE_DIM, be), lambda i: (0, i)),
            pl.BlockSpec((EDGE_DIM, HIDDEN), lambda i: (0, 0)),
            pl.BlockSpec((1, HIDDEN), lambda i: (0, 0)),
            pl.BlockSpec((2, be), lambda i: (0, i)),
        ],
        out_specs=[
            pl.BlockSpec((be * HIDDEN // 2,), lambda i: (i,)),
            pl.BlockSpec((be,), lambda i: (i,)),
            pl.BlockSpec((be,), lambda i: (i,)),
        ],
        out_shape=[
            jax.ShapeDtypeStruct((E_PAD * HIDDEN // 2,), jnp.int32),
            jax.ShapeDtypeStruct((E_PAD,), jnp.int32),
            jax.ShapeDtypeStruct((E_PAD,), jnp.int32),
        ],
    )(edge_attr.T, W_e, b_e.reshape(1, HIDDEN), edge_index)


def _linearize_body(x_ref, o_ref):
    o_ref[...] = x_ref[...].reshape(o_ref.shape)


def _linearize(x):
    bn = 1000
    grid = N_NODES // bn
    return pl.pallas_call(
        _linearize_body,
        grid=(grid,),
        in_specs=[pl.BlockSpec((bn, HIDDEN), lambda i: (i, 0))],
        out_specs=pl.BlockSpec((bn * HIDDEN,), lambda i: (i,)),
        out_shape=jax.ShapeDtypeStruct((N_NODES * HIDDEN,), jnp.float32),
    )(x)


def _sc_body(x_hbm, src_hbm, dst_hbm, e_hbm, out_hbm,
             srcb0, srcb1, dstb0, dstb1, ebuf0, ebuf1, xbuf0, xbuf1,
             agg, esem0, esem1, isem0, isem1, ssem0, ssem1):
    cid = lax.axis_index("c")
    sid = lax.axis_index("s")
    wid = cid * NS + sid
    base = wid * PER_TILE

    zero16 = jnp.zeros((16,), jnp.float32)

    def zfill(i, _):
        for k in range(HIDDEN // 16):
            xbuf0[i, pl.ds(k * 16, 16)] = zero16
        return 0

    lax.fori_loop(0, CHUNK, zfill, 0)
    for t in range(ROWS_PER_TILE // CHUNK):
        pltpu.sync_copy(
            xbuf0, agg.at[pl.ds(sid * ROWS_PER_TILE + t * CHUNK, CHUNK)]
        )
    plsc.subcore_barrier()

    bufs = (
        (ebuf0, xbuf0, srcb0, dstb0, esem0, isem0, ssem0),
        (ebuf1, xbuf1, srcb1, dstb1, esem1, isem1, ssem1),
    )
    ewords = wid * (HALF * HIDDEN)

    def voff(v):
        return base + (v % 2) * HALF + (v // 2) * CHUNK

    def issue_idx(v, b):
        sb, db, isem = bufs[b][2], bufs[b][3], bufs[b][5]
        off = voff(v)
        pltpu.async_copy(src_hbm.at[pl.ds(off, CHUNK)], sb, isem)
        pltpu.async_copy(dst_hbm.at[pl.ds(off, CHUNK)], db, isem)

    def wait_idx(b):
        sb, db, isem = bufs[b][2], bufs[b][3], bufs[b][5]
        pltpu.make_async_copy(src_hbm.at[pl.ds(0, CHUNK)], sb, isem).wait()
        pltpu.make_async_copy(src_hbm.at[pl.ds(0, CHUNK)], db, isem).wait()

    def issue_e(s, sb):
        eb, esem = bufs[sb][0], bufs[sb][4]
        pltpu.async_copy(
            e_hbm.at[pl.ds(ewords + s * SWORDS, SWORDS)], eb, esem)

    def wait_e(sb):
        eb, esem = bufs[sb][0], bufs[sb][4]
        pltpu.make_async_copy(e_hbm.at[pl.ds(0, SWORDS)], eb, esem).wait()

    def issue_x(v, b):
        xb, sb = bufs[b][1], bufs[b][2]
        pltpu.async_copy(x_hbm.at[sb], xb, bufs[b][5])

    def wait_x(b):
        xb = bufs[b][1]
        pltpu.make_async_copy(x_hbm.at[pl.ds(0, CHUNK)], xb,
                              bufs[b][5]).wait()

    def wait_scatter(b):
        xb, db, ssem = bufs[b][1], bufs[b][3], bufs[b][6]
        pltpu.make_async_copy(xb, agg.at[db], ssem).wait()

    def compute(h, sb, b):
        eb, xb, db, ssem = (bufs[sb][0], bufs[b][1], bufs[b][3],
                            bufs[b][6])

        def erow(i, _):
            for g in range(HIDDEN // 16):
                vi = eb[pl.ds(i * HIDDEN + g * 16, 16)]
                if h == 0:
                    ef = lax.bitcast_convert_type(vi << 16, jnp.float32)
                else:
                    ef = lax.bitcast_convert_type(
                        vi & jnp.int32(-65536), jnp.float32)
                sl = pl.ds(g * 16, 16)
                xb[i, sl] = jnp.maximum(ef + xb[i, sl], 0.0)
            return 0

        lax.fori_loop(0, CHUNK, erow, 0)
        pltpu.async_copy(xb, agg.at[db], ssem, add=True)

    issue_idx(0, 0)
    issue_idx(1, 1)
    issue_e(0, 0)
    wait_idx(0)
    issue_x(0, 0)

    def step(v, h, sb):
        b = h

        @pl.when(jnp.logical_and(v + 1 < NCHUNK, v >= 1))
        def _():
            wait_scatter(1 - b)

        @pl.when(v + 1 < NCHUNK)
        def _():
            wait_idx(1 - b)
            issue_x(v + 1, 1 - b)

        if h == 0:
            wait_e(sb)

            @pl.when(v + 2 < NCHUNK)
            def _():
                issue_e(v // 2 + 1, 1 - sb)

        wait_x(b)
        compute(h, sb, b)

        @pl.when(v + 2 < NCHUNK)
        def _():
            issue_idx(v + 2, b)

    def quad_body(j, _):
        for q in range(4):
            step(j * 4 + q, q % 2, q // 2)
        return 0

    lax.fori_loop(0, NCHUNK // 4, quad_body, 0)
    wait_scatter(0)
    wait_scatter(1)
    plsc.subcore_barrier()

    rb = pl.ds(sid * ROWS_PER_TILE, ROWS_PER_TILE)
    pltpu.sync_copy(agg.at[rb], out_hbm.at[cid].at[rb])


def _sc_aggregate(x, src3, dst3, e):
    mesh = plsc.VectorSubcoreMesh(core_axis_name="c", subcore_axis_name="s")
    k = pl.kernel(
        _sc_body,
        out_type=jax.ShapeDtypeStruct((NC, AGG_ROWS, HIDDEN), jnp.float32),
        mesh=mesh,
        scratch_types=[
            pltpu.VMEM((CHUNK,), jnp.int32),
            pltpu.VMEM((CHUNK,), jnp.int32),
            pltpu.VMEM((CHUNK,), jnp.int32),
            pltpu.VMEM((CHUNK,), jnp.int32),
            pltpu.VMEM((SWORDS,), jnp.int32),
            pltpu.VMEM((SWORDS,), jnp.int32),
            pltpu.VMEM((CHUNK, HIDDEN), jnp.float32),
            pltpu.VMEM((CHUNK, HIDDEN), jnp.float32),
            pltpu.VMEM_SHARED((AGG_ROWS, HIDDEN), jnp.float32),
            pltpu.SemaphoreType.DMA,
            pltpu.SemaphoreType.DMA,
            pltpu.SemaphoreType.DMA,
            pltpu.SemaphoreType.DMA,
            pltpu.SemaphoreType.DMA,
            pltpu.SemaphoreType.DMA,
        ],
    )
    return k(x, src3, dst3, e)


def _node_mlp_body(x_ref, a_ref, w1_ref, b1_ref, w2_ref, b2_ref,
                   g_ref, bt_ref, o_ref):
    x = x_ref[...]
    h = x + a_ref[0] + a_ref[1]
    h = jnp.maximum(
        jnp.dot(h, w1_ref[...], preferred_element_type=jnp.float32)
        + b1_ref[...], 0.0)
    h = (jnp.dot(h, w2_ref[...], preferred_element_type=jnp.float32)
         + b2_ref[...])
    mean = jnp.mean(h, axis=0, keepdims=True)
    var = jnp.mean((h - mean) ** 2, axis=0, keepdims=True)
    h = (h - mean) * lax.rsqrt(var + 1e-5) * g_ref[...] + bt_ref[...]
    o_ref[...] = jnp.maximum(h, 0.0) + x


def _node_mlp(x, aggs, W1, b1, W2, b2, gamma, beta):
    return pl.pallas_call(
        _node_mlp_body,
        grid=(1,),
        in_specs=[
            pl.BlockSpec((N_NODES, HIDDEN), lambda i: (0, 0)),
            pl.BlockSpec((NC, N_NODES, HIDDEN), lambda i: (0, 0, 0)),
            pl.BlockSpec((HIDDEN, HIDDEN), lambda i: (0, 0)),
            pl.BlockSpec((1, HIDDEN), lambda i: (0, 0)),
            pl.BlockSpec((HIDDEN, HIDDEN), lambda i: (0, 0)),
            pl.BlockSpec((1, HIDDEN), lambda i: (0, 0)),
            pl.BlockSpec((1, HIDDEN), lambda i: (0, 0)),
            pl.BlockSpec((1, HIDDEN), lambda i: (0, 0)),
        ],
        out_specs=pl.BlockSpec((N_NODES, HIDDEN), lambda i: (0, 0)),
        out_shape=jax.ShapeDtypeStruct((N_NODES, HIDDEN), jnp.float32),
    )(x, aggs, W1, b1.reshape(1, HIDDEN), W2, b2.reshape(1, HIDDEN),
      gamma.reshape(1, HIDDEN), beta.reshape(1, HIDDEN))


def kernel(x, edge_index, edge_attr, W_e, b_e, W1, b1, W2, b2, gamma, beta):
    e, src, dst = _edge_mlp(edge_attr, W_e, b_e,
                            edge_index.astype(jnp.int32))
    x_lin = _linearize(x).reshape(N_NODES, HIDDEN)
    aggs = _sc_aggregate(x_lin, src, dst, e)
    return _node_mlp(x, aggs, W1, b1, W2, b2, gamma, beta)
